# baseline (speedup 1.0000x reference)
import functools

import jax
import jax.numpy as jnp
from jax import lax
from jax.experimental import pallas as pl
from jax.experimental.pallas import tpu as pltpu
from jax.experimental.pallas import tpu_sc as plsc

THRESHOLD = 0.5
N, H, W = 5000, 64, 64
L = 16
NC, NS = 2, 16
NW = NC * NS
NPAD = 5120
RB = 2
G = H // RB
CHL = 128
NCH = NPAD // CHL
SC_ITERS = -(-NCH // NW)
NG = CHL // L


def _tc_body(x_ref, cp_ref, rm_ref, cm_ref):
    x = x_ref[...]
    cp_ref[...] = x
    sub = pl.program_id(0) % (8 // RB)
    part = jnp.max(x, axis=1)
    full = jnp.concatenate([part] * (8 // RB), axis=0)
    rows = lax.broadcasted_iota(jnp.int32, (8, NPAD), 0)
    band = (rows // RB) == sub
    rm_ref[...] = jnp.where(band, full, rm_ref[...])

    @pl.when(pl.program_id(0) == 0)
    def _init():
        cm_ref[...] = jnp.full((W, NPAD), -jnp.inf, jnp.float32)

    cm_ref[...] = jnp.maximum(cm_ref[...], jnp.max(x, axis=0))


_tc_pass = pl.pallas_call(
    _tc_body,
    grid=(G,),
    in_specs=[pl.BlockSpec((RB, W, NPAD), lambda g: (g, 0, 0))],
    out_specs=[
        pl.BlockSpec((RB, W, NPAD), lambda g: (g, 0, 0)),
        pl.BlockSpec((8, NPAD), lambda g: (g // (8 // RB), 0)),
        pl.BlockSpec((W, NPAD), lambda g: (0, 0)),
    ],
    out_shape=[
        jax.ShapeDtypeStruct((H, W, N), jnp.float32),
        jax.ShapeDtypeStruct((H, NPAD), jnp.float32),
        jax.ShapeDtypeStruct((W, NPAD), jnp.float32),
    ],
)

_mesh = plsc.VectorSubcoreMesh(core_axis_name="c", subcore_axis_name="s")


@functools.partial(
    pl.kernel,
    mesh=_mesh,
    out_type=jax.ShapeDtypeStruct((4, NPAD), jnp.float32),
    scratch_types=[
        pltpu.VMEM((H, CHL), jnp.float32),
        pltpu.VMEM((W, CHL), jnp.float32),
        pltpu.VMEM((4, CHL), jnp.float32),
    ],
    compiler_params=pltpu.CompilerParams(needs_layout_passes=False),
)
def _sc_boxes(rm_hbm, cm_hbm, out_hbm, rbuf, cbuf, obuf):
    wid = lax.axis_index("s") * NC + lax.axis_index("c")
    h_full = jnp.full((L,), float(H), jnp.float32)
    neg1 = jnp.full((L,), -1.0, jnp.float32)

    for i in range(SC_ITERS):
        cid = i * NW + wid

        @pl.when(cid < NCH)
        def _process():
            base = cid * CHL
            pltpu.sync_copy(rm_hbm.at[:, pl.ds(base, CHL)], rbuf)
            pltpu.sync_copy(cm_hbm.at[:, pl.ds(base, CHL)], cbuf)

            for g in range(NG):
                def body(r, carry):
                    mnr, mxr, mnc, mxc = carry
                    vr = rbuf[r, pl.ds(g * L, L)]
                    vc = cbuf[r, pl.ds(g * L, L)]
                    rf = r.astype(jnp.float32)
                    br = vr > THRESHOLD
                    bc = vc > THRESHOLD
                    mnr = jnp.minimum(mnr, jnp.where(br, rf, float(H)))
                    mxr = jnp.maximum(mxr, jnp.where(br, rf, -1.0))
                    mnc = jnp.minimum(mnc, jnp.where(bc, rf, float(W)))
                    mxc = jnp.maximum(mxc, jnp.where(bc, rf, -1.0))
                    return mnr, mxr, mnc, mxc

                mnr, mxr, mnc, mxc = lax.fori_loop(
                    0, H, body, (h_full, neg1, h_full, neg1), unroll=8)

                obuf[0, pl.ds(g * L, L)] = mnc
                obuf[1, pl.ds(g * L, L)] = mnr
                obuf[2, pl.ds(g * L, L)] = mxc
                obuf[3, pl.ds(g * L, L)] = mxr

            pltpu.sync_copy(obuf, out_hbm.at[:, pl.ds(base, CHL)])


def kernel(masks):
    mt = jnp.transpose(masks, (1, 2, 0))
    cp, rm, cm = _tc_pass(mt)
    b4 = _sc_boxes(rm, cm)
    masks_out = jnp.transpose(cp, (2, 0, 1))
    boxes_2d = jnp.transpose(b4[:, :N]).reshape(N, 2, 2)
    return masks_out, boxes_2d


# speedup vs baseline: 1.0564x; 1.0564x over previous
import functools

import jax
import jax.numpy as jnp
from jax import lax
from jax.experimental import pallas as pl
from jax.experimental.pallas import tpu as pltpu
from jax.experimental.pallas import tpu_sc as plsc

THRESHOLD = 0.5
N, H, W = 5000, 64, 64
L = 16
NC, NS = 2, 16
NW = NC * NS
BN = 512
G = -(-N // BN)
NPAD = G * BN
CHL = 128
NCH = NPAD // CHL
SC_ITERS = -(-NCH // NW)
NG = CHL // L


def _tc_body(x_ref, cp_ref, rm_ref, cm_ref):
    x = x_ref[...]
    cp_ref[...] = x
    rm_ref[...] = jnp.max(x, axis=1)
    cm_ref[...] = jnp.max(x, axis=0)


_tc_pass = pl.pallas_call(
    _tc_body,
    grid=(G,),
    in_specs=[pl.BlockSpec((H, W, BN), lambda g: (0, 0, g))],
    out_specs=[
        pl.BlockSpec((H, W, BN), lambda g: (0, 0, g)),
        pl.BlockSpec((H, BN), lambda g: (0, g)),
        pl.BlockSpec((W, BN), lambda g: (0, g)),
    ],
    out_shape=[
        jax.ShapeDtypeStruct((H, W, N), jnp.float32),
        jax.ShapeDtypeStruct((H, NPAD), jnp.float32),
        jax.ShapeDtypeStruct((W, NPAD), jnp.float32),
    ],
    compiler_params=pltpu.CompilerParams(vmem_limit_bytes=50 * 1024 * 1024),
)

_mesh = plsc.VectorSubcoreMesh(core_axis_name="c", subcore_axis_name="s")


@functools.partial(
    pl.kernel,
    mesh=_mesh,
    out_type=jax.ShapeDtypeStruct((4, NPAD), jnp.float32),
    scratch_types=[
        pltpu.VMEM((H, CHL), jnp.float32),
        pltpu.VMEM((W, CHL), jnp.float32),
        pltpu.VMEM((4, CHL), jnp.float32),
    ],
    compiler_params=pltpu.CompilerParams(needs_layout_passes=False),
)
def _sc_boxes(rm_hbm, cm_hbm, out_hbm, rbuf, cbuf, obuf):
    wid = lax.axis_index("s") * NC + lax.axis_index("c")
    h_full = jnp.full((L,), float(H), jnp.float32)
    neg1 = jnp.full((L,), -1.0, jnp.float32)

    for i in range(SC_ITERS):
        cid = i * NW + wid

        @pl.when(cid < NCH)
        def _process():
            base = cid * CHL
            pltpu.sync_copy(rm_hbm.at[:, pl.ds(base, CHL)], rbuf)
            pltpu.sync_copy(cm_hbm.at[:, pl.ds(base, CHL)], cbuf)

            for g in range(NG):
                def body(r, carry):
                    mnr, mxr, mnc, mxc = carry
                    vr = rbuf[r, pl.ds(g * L, L)]
                    vc = cbuf[r, pl.ds(g * L, L)]
                    rf = r.astype(jnp.float32)
                    br = vr > THRESHOLD
                    bc = vc > THRESHOLD
                    mnr = jnp.minimum(mnr, jnp.where(br, rf, float(H)))
                    mxr = jnp.maximum(mxr, jnp.where(br, rf, -1.0))
                    mnc = jnp.minimum(mnc, jnp.where(bc, rf, float(W)))
                    mxc = jnp.maximum(mxc, jnp.where(bc, rf, -1.0))
                    return mnr, mxr, mnc, mxc

                mnr, mxr, mnc, mxc = lax.fori_loop(
                    0, H, body, (h_full, neg1, h_full, neg1), unroll=8)

                obuf[0, pl.ds(g * L, L)] = mnc
                obuf[1, pl.ds(g * L, L)] = mnr
                obuf[2, pl.ds(g * L, L)] = mxc
                obuf[3, pl.ds(g * L, L)] = mxr

            pltpu.sync_copy(obuf, out_hbm.at[:, pl.ds(base, CHL)])


def kernel(masks):
    mt = jnp.transpose(masks, (1, 2, 0))
    cp, rm, cm = _tc_pass(mt)
    b4 = _sc_boxes(rm, cm)
    masks_out = jnp.transpose(cp, (2, 0, 1))
    boxes_2d = jnp.transpose(b4[:, :N]).reshape(N, 2, 2)
    return masks_out, boxes_2d


# speedup vs baseline: 1.1046x; 1.0456x over previous
import functools

import jax
import jax.numpy as jnp
from jax import lax
from jax.experimental import pallas as pl
from jax.experimental.pallas import tpu as pltpu
from jax.experimental.pallas import tpu_sc as plsc

THRESHOLD = 0.5
N, H, W = 5000, 64, 64
L = 16
NC, NS = 2, 16
NW = NC * NS
BN = 512
G = -(-N // BN)
NPAD = G * BN
CHL = 128
NCH = NPAD // CHL
SC_ITERS = -(-NCH // NW)
NG = CHL // L


def _tc_body(x_ref, cp_ref, tbl_ref):
    x = x_ref[...]
    cp_ref[...] = x
    tbl_ref[pl.ds(0, H), :] = jnp.max(x, axis=1)
    tbl_ref[pl.ds(H, W), :] = jnp.max(x, axis=0)


_tc_pass = pl.pallas_call(
    _tc_body,
    grid=(G,),
    in_specs=[pl.BlockSpec((H, W, BN), lambda g: (0, 0, g))],
    out_specs=[
        pl.BlockSpec((H, W, BN), lambda g: (0, 0, g)),
        pl.BlockSpec((H + W, BN), lambda g: (0, g)),
    ],
    out_shape=[
        jax.ShapeDtypeStruct((H, W, N), jnp.float32),
        jax.ShapeDtypeStruct((H + W, NPAD), jnp.float32),
    ],
    compiler_params=pltpu.CompilerParams(vmem_limit_bytes=50 * 1024 * 1024),
)

_mesh = plsc.VectorSubcoreMesh(core_axis_name="c", subcore_axis_name="s")


@functools.partial(
    pl.kernel,
    mesh=_mesh,
    out_type=jax.ShapeDtypeStruct((2, 2, NPAD), jnp.float32),
    scratch_types=[
        pltpu.VMEM((H + W, CHL), jnp.float32),
        pltpu.VMEM((2, 2, CHL), jnp.float32),
    ],
    compiler_params=pltpu.CompilerParams(needs_layout_passes=False),
)
def _sc_boxes(tbl_hbm, out_hbm, buf, obuf):
    wid = lax.axis_index("s") * NC + lax.axis_index("c")
    h_full = jnp.full((L,), float(H), jnp.float32)
    neg1 = jnp.full((L,), -1.0, jnp.float32)

    def chunk_body(i, _):
        cid = i * NW + wid

        @pl.when(cid < NCH)
        def _process():
            base = cid * CHL
            pltpu.sync_copy(tbl_hbm.at[:, pl.ds(base, CHL)], buf)

            def group_body(g, _):
                off = pl.multiple_of(g * L, L)

                def body(r, carry):
                    mnr, mxr, mnc, mxc = carry
                    vr = buf[r, pl.ds(off, L)]
                    vc = buf[r + H, pl.ds(off, L)]
                    rf = r.astype(jnp.float32)
                    br = vr > THRESHOLD
                    bc = vc > THRESHOLD
                    mnr = jnp.minimum(mnr, jnp.where(br, rf, float(H)))
                    mxr = jnp.maximum(mxr, jnp.where(br, rf, -1.0))
                    mnc = jnp.minimum(mnc, jnp.where(bc, rf, float(W)))
                    mxc = jnp.maximum(mxc, jnp.where(bc, rf, -1.0))
                    return mnr, mxr, mnc, mxc

                mnr, mxr, mnc, mxc = lax.fori_loop(
                    0, H, body, (h_full, neg1, h_full, neg1), unroll=4)

                obuf[0, 0, pl.ds(off, L)] = mnc
                obuf[0, 1, pl.ds(off, L)] = mnr
                obuf[1, 0, pl.ds(off, L)] = mxc
                obuf[1, 1, pl.ds(off, L)] = mxr
                return 0

            lax.fori_loop(0, NG, group_body, 0)
            pltpu.sync_copy(obuf, out_hbm.at[:, :, pl.ds(base, CHL)])

        return 0

    lax.fori_loop(0, SC_ITERS, chunk_body, 0)


def kernel(masks):
    mt = jnp.transpose(masks, (1, 2, 0))
    cp, tbl = _tc_pass(mt)
    b4 = _sc_boxes(tbl)
    masks_out = jnp.transpose(cp, (2, 0, 1))
    boxes_2d = jnp.transpose(b4[:, :, :N], (2, 0, 1))
    return masks_out, boxes_2d


# speedup vs baseline: 1.1544x; 1.0451x over previous
import functools

import jax
import jax.numpy as jnp
from jax import lax
from jax.experimental import pallas as pl
from jax.experimental.pallas import tpu as pltpu
from jax.experimental.pallas import tpu_sc as plsc

THRESHOLD = 0.5
N, H, W = 5000, 64, 64
L = 16
NC, NS = 2, 16
NW = NC * NS
BN = 512
G = -(-N // BN)
NPAD = G * BN
CHL = 128
NCH = NPAD // CHL
SC_ITERS = -(-NCH // NW)
NG = CHL // L


def _tc_body(x_ref, cp_ref, tbl_ref):
    x = x_ref[...]
    cp_ref[...] = x
    rm = jnp.max(x, axis=1) > THRESHOLD
    cm = jnp.max(x, axis=0) > THRESHOLD
    w32 = jnp.int32(1) << lax.broadcasted_iota(jnp.int32, (32, BN), 0)

    def bits(b):
        lo = jnp.sum(jnp.where(b[:32], w32, 0), axis=0)
        hi = jnp.sum(jnp.where(b[32:], w32, 0), axis=0)
        return lo, hi

    rlo, rhi = bits(rm)
    clo, chi = bits(cm)
    z = jnp.zeros_like(rlo)
    tbl_ref[...] = jnp.stack([rlo, rhi, clo, chi, z, z, z, z], axis=0)


_tc_pass = pl.pallas_call(
    _tc_body,
    grid=(G,),
    in_specs=[pl.BlockSpec((H, W, BN), lambda g: (0, 0, g))],
    out_specs=[
        pl.BlockSpec((H, W, BN), lambda g: (0, 0, g)),
        pl.BlockSpec((8, BN), lambda g: (0, g)),
    ],
    out_shape=[
        jax.ShapeDtypeStruct((H, W, N), jnp.float32),
        jax.ShapeDtypeStruct((8, NPAD), jnp.int32),
    ],
    compiler_params=pltpu.CompilerParams(vmem_limit_bytes=50 * 1024 * 1024),
)


def _ctz(x):
    n = jnp.zeros((L,), jnp.int32)
    for shift, mask in ((16, 0xFFFF), (8, 0xFF), (4, 0xF), (2, 0x3), (1, 0x1)):
        c = (x & mask) == 0
        n = n + jnp.where(c, shift, 0)
        x = jnp.where(c, x >> shift, x)
    return n


def _fls(x):
    n = jnp.zeros((L,), jnp.int32)
    for shift, mask in ((16, -65536), (8, 0xFF00), (4, 0xF0), (2, 0xC), (1, 0x2)):
        c = (x & mask) != 0
        n = n + jnp.where(c, shift, 0)
        x = jnp.where(c, x >> shift, x)
    return n


def _minmax_idx(lo, hi, empty_min, empty_max):
    lo0 = lo == 0
    hi0 = hi == 0
    both0 = lo0 & hi0
    mn = jnp.where(both0, empty_min,
                   jnp.where(lo0, 32 + _ctz(hi), _ctz(lo)))
    mx = jnp.where(both0, empty_max,
                   jnp.where(hi0, _fls(lo), 32 + _fls(hi)))
    return mn.astype(jnp.float32), mx.astype(jnp.float32)

_mesh = plsc.VectorSubcoreMesh(core_axis_name="c", subcore_axis_name="s")


@functools.partial(
    pl.kernel,
    mesh=_mesh,
    out_type=jax.ShapeDtypeStruct((2, 2, NPAD), jnp.float32),
    scratch_types=[
        pltpu.VMEM((8, CHL), jnp.int32),
        pltpu.VMEM((2, 2, CHL), jnp.float32),
    ],
    compiler_params=pltpu.CompilerParams(needs_layout_passes=False),
)
def _sc_boxes(tbl_hbm, out_hbm, buf, obuf):
    wid = lax.axis_index("s") * NC + lax.axis_index("c")

    def chunk_body(i, _):
        cid = i * NW + wid

        @pl.when(cid < NCH)
        def _process():
            base = cid * CHL
            pltpu.sync_copy(tbl_hbm.at[:, pl.ds(base, CHL)], buf)

            def group_body(g, _):
                off = pl.multiple_of(g * L, L)
                rlo = buf[0, pl.ds(off, L)]
                rhi = buf[1, pl.ds(off, L)]
                clo = buf[2, pl.ds(off, L)]
                chi = buf[3, pl.ds(off, L)]
                mnr, mxr = _minmax_idx(rlo, rhi, H, -1)
                mnc, mxc = _minmax_idx(clo, chi, W, -1)
                obuf[0, 0, pl.ds(off, L)] = mnc
                obuf[0, 1, pl.ds(off, L)] = mnr
                obuf[1, 0, pl.ds(off, L)] = mxc
                obuf[1, 1, pl.ds(off, L)] = mxr
                return 0

            lax.fori_loop(0, NG, group_body, 0)
            pltpu.sync_copy(obuf, out_hbm.at[:, :, pl.ds(base, CHL)])

        return 0

    lax.fori_loop(0, SC_ITERS, chunk_body, 0)


def kernel(masks):
    mt = jnp.transpose(masks, (1, 2, 0))
    cp, tbl = _tc_pass(mt)
    b4 = _sc_boxes(tbl)
    masks_out = jnp.transpose(cp, (2, 0, 1))
    boxes_2d = jnp.transpose(b4[:, :, :N], (2, 0, 1))
    return masks_out, boxes_2d


# speedup vs baseline: 1.1666x; 1.0106x over previous
import functools

import jax
import jax.numpy as jnp
from jax import lax
from jax.experimental import pallas as pl
from jax.experimental.pallas import tpu as pltpu
from jax.experimental.pallas import tpu_sc as plsc

THRESHOLD = 0.5
N, H, W = 5000, 64, 64
L = 16
NC, NS = 2, 16
NW = NC * NS
BN = 512
G = -(-N // BN)
NPAD = G * BN
CHL = 256
NCH = NPAD // CHL
NG = CHL // L


def _tc_body(x_ref, cp_ref, tbl_ref):
    x = x_ref[...]
    cp_ref[...] = x
    rm = jnp.max(x, axis=1) > THRESHOLD
    cm = jnp.max(x, axis=0) > THRESHOLD
    w32 = jnp.int32(1) << lax.broadcasted_iota(jnp.int32, (32, BN), 0)

    def bits(b):
        lo = jnp.sum(jnp.where(b[:32], w32, 0), axis=0)
        hi = jnp.sum(jnp.where(b[32:], w32, 0), axis=0)
        return lo, hi

    rlo, rhi = bits(rm)
    clo, chi = bits(cm)
    z = jnp.zeros_like(rlo)
    tbl_ref[...] = jnp.stack([rlo, rhi, clo, chi, z, z, z, z], axis=0)


_tc_pass = pl.pallas_call(
    _tc_body,
    grid=(G,),
    in_specs=[pl.BlockSpec((H, W, BN), lambda g: (0, 0, g))],
    out_specs=[
        pl.BlockSpec((H, W, BN), lambda g: (0, 0, g)),
        pl.BlockSpec((8, BN), lambda g: (0, g)),
    ],
    out_shape=[
        jax.ShapeDtypeStruct((H, W, N), jnp.float32),
        jax.ShapeDtypeStruct((8, NPAD), jnp.int32),
    ],
    compiler_params=pltpu.CompilerParams(vmem_limit_bytes=50 * 1024 * 1024),
)


def _ctz(x):
    n = jnp.zeros((L,), jnp.int32)
    for shift, mask in ((16, 0xFFFF), (8, 0xFF), (4, 0xF), (2, 0x3), (1, 0x1)):
        c = (x & mask) == 0
        n = n + jnp.where(c, shift, 0)
        x = jnp.where(c, x >> shift, x)
    return n


def _fls(x):
    n = jnp.zeros((L,), jnp.int32)
    for shift, mask in ((16, -65536), (8, 0xFF00), (4, 0xF0), (2, 0xC), (1, 0x2)):
        c = (x & mask) != 0
        n = n + jnp.where(c, shift, 0)
        x = jnp.where(c, x >> shift, x)
    return n


def _minmax_idx(lo, hi, empty_min, empty_max):
    lo0 = lo == 0
    hi0 = hi == 0
    both0 = lo0 & hi0
    mn = jnp.where(both0, empty_min,
                   jnp.where(lo0, 32 + _ctz(hi), _ctz(lo)))
    mx = jnp.where(both0, empty_max,
                   jnp.where(hi0, _fls(lo), 32 + _fls(hi)))
    return mn.astype(jnp.float32), mx.astype(jnp.float32)

_mesh = plsc.VectorSubcoreMesh(core_axis_name="c", subcore_axis_name="s")


@functools.partial(
    pl.kernel,
    mesh=_mesh,
    out_type=jax.ShapeDtypeStruct((2, 2, NPAD), jnp.float32),
    scratch_types=[
        pltpu.VMEM((8, CHL), jnp.int32),
        pltpu.VMEM((2, 2, CHL), jnp.float32),
    ],
    compiler_params=pltpu.CompilerParams(needs_layout_passes=False),
)
def _sc_boxes(tbl_hbm, out_hbm, buf, obuf):
    wid = lax.axis_index("s") * NC + lax.axis_index("c")
    cid = wid

    @pl.when(cid < NCH)
    def _process():
        base = cid * CHL
        pltpu.sync_copy(tbl_hbm.at[:, pl.ds(base, CHL)], buf)

        def group_body(g, _):
            off = pl.multiple_of(g * L, L)
            rlo = buf[0, pl.ds(off, L)]
            rhi = buf[1, pl.ds(off, L)]
            clo = buf[2, pl.ds(off, L)]
            chi = buf[3, pl.ds(off, L)]
            mnr, mxr = _minmax_idx(rlo, rhi, H, -1)
            mnc, mxc = _minmax_idx(clo, chi, W, -1)
            obuf[0, 0, pl.ds(off, L)] = mnc
            obuf[0, 1, pl.ds(off, L)] = mnr
            obuf[1, 0, pl.ds(off, L)] = mxc
            obuf[1, 1, pl.ds(off, L)] = mxr
            return 0

        lax.fori_loop(0, NG, group_body, 0)
        pltpu.sync_copy(obuf, out_hbm.at[:, :, pl.ds(base, CHL)])


def kernel(masks):
    mt = jnp.transpose(masks, (1, 2, 0))
    cp, tbl = _tc_pass(mt)
    b4 = _sc_boxes(tbl)
    masks_out = jnp.transpose(cp, (2, 0, 1))
    boxes_2d = jnp.transpose(b4[:, :, :N], (2, 0, 1))
    return masks_out, boxes_2d
